# baseline (speedup 1.0000x reference)
import jax
import jax.numpy as jnp
from jax.experimental import pallas as pl
from jax.experimental.pallas import tpu as pltpu

_H, _W, _C = 38, 50, 512
_HW = _H * _W
_TR = _HW * 4
_NR = 8
_NSTEP = 16


def _tr_kernel(src, out):
    for q in range(4):
        out[0, :, q, :] = jnp.transpose(
            src[0, q * 128:(q + 1) * 128, :], (1, 0))


def _pool_kernel(yidx, xidx, feat, out, tbl):
    s = pl.program_id(1)

    @pl.when(s == 0)
    def _build():
        def shmax(dst, src, shift, n):
            tbl[dst:dst + n] = jnp.maximum(
                tbl[src:src + n], tbl[src + shift:src + shift + n])
        tbl[0:_HW] = feat[0]
        shmax(_HW, 0, 1, _HW - 1)
        shmax(2 * _HW, _HW, 2, _HW - 2)
        for kx in range(3):
            shmax((3 + kx) * _HW, kx * _HW, 50, _HW - 50)
        for kx in range(3):
            shmax((6 + kx) * _HW, (3 + kx) * _HW, 100, _HW - 100)

    for rr in range(_NR):
        r14 = (s * _NR + rr) * 14
        ya = [yidx[0, 0, r14 + 2 * i] for i in range(7)]
        yb = [yidx[0, 0, r14 + 2 * i + 1] for i in range(7)]
        xa = [xidx[0, 0, r14 + 2 * j] for j in range(7)]
        xb = [xidx[0, 0, r14 + 2 * j + 1] for j in range(7)]
        for i in range(7):
            for j in range(7):
                v = jnp.maximum(
                    jnp.maximum(tbl[ya[i] + xa[j]], tbl[ya[i] + xb[j]]),
                    jnp.maximum(tbl[yb[i] + xa[j]], tbl[yb[i] + xb[j]]))
                out[i * 7 + j, 0, rr, :, :] = v


def _mm1_kernel(x, w1, b1r, out, acc):
    k = pl.program_id(1)

    @pl.when(k == 0)
    def _():
        acc[...] = jnp.zeros_like(acc)

    acc[...] += jnp.dot(x[...], w1[...], preferred_element_type=jnp.float32)

    @pl.when(k == 13)
    def _():
        out[...] = acc[...] + b1r[...]


def _mm2_kernel(h1b, w2, b2r, wbb, wsb, ocb, ocs, acc):
    k = pl.program_id(1)

    @pl.when(k == 0)
    def _():
        acc[...] = jnp.zeros_like(acc)

    acc[...] += jnp.dot(h1b[...], w2[...],
                        preferred_element_type=jnp.float32)

    @pl.when(k == 3)
    def _():
        x2h = acc[...] + b2r[...]
        ocb[0] = jnp.dot(x2h, wbb[...], preferred_element_type=jnp.float32)
        ocs[0] = jnp.dot(x2h, wsb[...], preferred_element_type=jnp.float32)


def _loss_kernel(pcb, pcs, bbr, bsr, onehotv, selpos, gt84, out):
    cb = pcb[0] + pcb[1] + bbr[...]
    cs = pcs[0] + pcs[1] + bsr[...]
    m = jnp.max(cs, axis=-1, keepdims=True)
    sh = cs - m
    lse = jnp.log(jnp.sum(jnp.exp(sh), axis=-1, keepdims=True))
    logp = sh - lse
    cn = jnp.sum(onehotv[...] * (-logp), axis=(0, 1), keepdims=True)
    cd = jnp.maximum(jnp.sum(onehotv[...], axis=(0, 1), keepdims=True), 1.0)
    cls_loss = cn / cd
    d = jnp.abs(gt84[...] - cb)
    sl1 = jnp.where(d < 1.0, 0.5 * d * d, d - 0.5)
    bbox_loss = jnp.sum(sl1 * selpos[...], axis=(0, 1), keepdims=True)
    out[...] = jnp.concatenate(
        [cls_loss, bbox_loss, cls_loss + 10.0 * bbox_loss], axis=1)


def kernel(base_feature, rois, gt_roi_label, gt_roi_bbox,
           W1, b1, W2, b2, Wb, bb, Ws, bs):
    B, C, H, W = base_feature.shape
    R = rois.shape[1]
    M = B * R
    K1 = Wb.shape[1]
    K = Ws.shape[1]

    feat = pl.pallas_call(
        _tr_kernel,
        grid=(B,),
        in_specs=[pl.BlockSpec((1, C, H * W), lambda b: (b, 0, 0))],
        out_specs=pl.BlockSpec((1, _HW, 4, 128), lambda b: (b, 0, 0, 0)),
        out_shape=jax.ShapeDtypeStruct((B, _HW, 4, 128), jnp.float32),
        compiler_params=pltpu.CompilerParams(
            dimension_semantics=("parallel",),
            vmem_limit_bytes=32 * 1024 * 1024),
    )(base_feature.reshape(B, C, H * W))

    x1 = rois[..., 0]
    y1 = rois[..., 1]
    x2 = rois[..., 2]
    y2 = rois[..., 3]
    w = x2 - x1 + 1
    h = y2 - y1 + 1
    ii = jnp.arange(7, dtype=jnp.int32)

    rs = y1[..., None] + (ii * h[..., None]) // 7
    re = y1[..., None] + ((ii + 1) * h[..., None] + 6) // 7
    hb = re - rs
    ky = (hb > 2).astype(jnp.int32) + (hb > 4).astype(jnp.int32)
    ya = ky * (3 * _HW) + rs * 50
    yb = ky * (3 * _HW) + (re - (1 << ky)) * 50
    yidx = jnp.stack([ya, yb], axis=-1).astype(jnp.int32).reshape(B, 1, R * 14)

    cs_ = x1[..., None] + (ii * w[..., None]) // 7
    ce_ = x1[..., None] + ((ii + 1) * w[..., None] + 6) // 7
    wb = ce_ - cs_
    kx = (wb > 2).astype(jnp.int32) + (wb > 4).astype(jnp.int32)
    xa = kx * _HW + cs_
    xb = kx * _HW + (ce_ - (1 << kx))
    xidx = jnp.stack([xa, xb], axis=-1).astype(jnp.int32).reshape(B, 1, R * 14)

    pooled = pl.pallas_call(
        _pool_kernel,
        grid=(B, _NSTEP),
        in_specs=[
            pl.BlockSpec((1, 1, R * 14), lambda b, s: (b, 0, 0),
                         memory_space=pltpu.SMEM),
            pl.BlockSpec((1, 1, R * 14), lambda b, s: (b, 0, 0),
                         memory_space=pltpu.SMEM),
            pl.BlockSpec((1, _HW, 4, 128), lambda b, s: (b, 0, 0, 0)),
        ],
        out_specs=pl.BlockSpec((49, 1, _NR, 4, 128),
                               lambda b, s: (0, b, s, 0, 0)),
        out_shape=jax.ShapeDtypeStruct((49, B, R, 4, 128), jnp.float32),
        scratch_shapes=[pltpu.VMEM((9 * _HW, 4, 128), jnp.float32)],
        compiler_params=pltpu.CompilerParams(
            dimension_semantics=("parallel", "arbitrary"),
            vmem_limit_bytes=58 * 1024 * 1024),
    )(yidx, xidx, feat)

    xf = pooled.transpose(1, 2, 3, 4, 0).reshape(M, 49 * 512)
    b1r = b1.reshape(1, 4096)

    h1 = pl.pallas_call(
        _mm1_kernel,
        grid=(2, 14),
        in_specs=[
            pl.BlockSpec((M, 1792), lambda nh, k: (0, k)),
            pl.BlockSpec((1792, 2048), lambda nh, k: (k, nh)),
            pl.BlockSpec((1, 2048), lambda nh, k: (0, nh)),
        ],
        out_specs=pl.BlockSpec((M, 2048), lambda nh, k: (0, nh)),
        out_shape=jax.ShapeDtypeStruct((M, 4096), jnp.float32),
        scratch_shapes=[pltpu.VMEM((M, 2048), jnp.float32)],
        compiler_params=pltpu.CompilerParams(
            dimension_semantics=("parallel", "arbitrary"),
            vmem_limit_bytes=50 * 1024 * 1024),
    )(xf, W1, b1r)

    b2r = b2.reshape(1, 4096)
    pcb, pcs = pl.pallas_call(
        _mm2_kernel,
        grid=(2, 4),
        in_specs=[
            pl.BlockSpec((M, 1024), lambda nh, k: (0, k)),
            pl.BlockSpec((1024, 2048), lambda nh, k: (k, nh)),
            pl.BlockSpec((1, 2048), lambda nh, k: (0, nh)),
            pl.BlockSpec((2048, K1), lambda nh, k: (nh, 0)),
            pl.BlockSpec((2048, K), lambda nh, k: (nh, 0)),
        ],
        out_specs=[
            pl.BlockSpec((1, M, K1), lambda nh, k: (nh, 0, 0)),
            pl.BlockSpec((1, M, K), lambda nh, k: (nh, 0, 0)),
        ],
        out_shape=[
            jax.ShapeDtypeStruct((2, M, K1), jnp.float32),
            jax.ShapeDtypeStruct((2, M, K), jnp.float32),
        ],
        scratch_shapes=[pltpu.VMEM((M, 2048), jnp.float32)],
        compiler_params=pltpu.CompilerParams(
            dimension_semantics=("parallel", "arbitrary"),
            vmem_limit_bytes=48 * 1024 * 1024),
    )(h1, W2, b2r, Wb, Ws)

    lblf = gt_roi_label.reshape(M)
    valid = (lblf != -1)
    lbl = jnp.where(valid, lblf, 0)
    onehot = (lbl[:, None] == jnp.arange(K, dtype=jnp.int32)[None, :])
    onehotv = (onehot & valid[:, None]).astype(jnp.float32)
    pos = (lblf > 0).astype(jnp.float32)
    selpos = jnp.repeat(onehot.astype(jnp.float32), 4, axis=1) * pos[:, None]
    gt84 = jnp.tile(gt_roi_bbox.reshape(M, 4), (1, K))

    out3 = pl.pallas_call(
        _loss_kernel,
        grid=(1,),
        in_specs=[
            pl.BlockSpec((2, M, K1), lambda i: (0, 0, 0)),
            pl.BlockSpec((2, M, K), lambda i: (0, 0, 0)),
            pl.BlockSpec((1, K1), lambda i: (0, 0)),
            pl.BlockSpec((1, K), lambda i: (0, 0)),
            pl.BlockSpec((M, K), lambda i: (0, 0)),
            pl.BlockSpec((M, K1), lambda i: (0, 0)),
            pl.BlockSpec((M, K1), lambda i: (0, 0)),
        ],
        out_specs=pl.BlockSpec((1, 3), lambda i: (0, 0)),
        out_shape=jax.ShapeDtypeStruct((1, 3), jnp.float32),
        compiler_params=pltpu.CompilerParams(
            dimension_semantics=("arbitrary",)),
    )(pcb, pcs, bb.reshape(1, K1), bs.reshape(1, K), onehotv, selpos, gt84)

    return out3[0]


# speedup vs baseline: 1.0098x; 1.0098x over previous
import jax
import jax.numpy as jnp
from jax.experimental import pallas as pl
from jax.experimental.pallas import tpu as pltpu

_H, _W, _C = 38, 50, 512
_HW = _H * _W
_TR = _HW * 4
_NR = 16
_NSTEP = 8


def _tr_kernel(src, out):
    for q in range(4):
        for y in range(_H):
            out[0, y * _W:(y + 1) * _W, q, :] = jnp.transpose(
                src[0, q * 128:(q + 1) * 128, y, :], (1, 0))


def _pool_kernel(yidx, xidx, feat, out, tbl):
    s = pl.program_id(1)

    @pl.when(s == 0)
    def _build():
        def shmax(dst, src, shift, n):
            tbl[dst:dst + n, :] = jnp.maximum(
                tbl[src:src + n, :], tbl[src + shift:src + shift + n, :])
        tbl[0:_TR, :] = feat[0]
        shmax(_TR, 0, 4, _TR - 4)
        shmax(2 * _TR, _TR, 8, _TR - 8)
        for kx in range(3):
            shmax((3 + kx) * _TR, kx * _TR, 200, _TR - 200)
        for kx in range(3):
            shmax((6 + kx) * _TR, (3 + kx) * _TR, 400, _TR - 400)

    for rr in range(_NR):
        r14 = (s * _NR + rr) * 14
        ya = [yidx[0, 0, r14 + 2 * i] for i in range(7)]
        yb = [yidx[0, 0, r14 + 2 * i + 1] for i in range(7)]
        xa = [xidx[0, 0, r14 + 2 * j] for j in range(7)]
        xb = [xidx[0, 0, r14 + 2 * j + 1] for j in range(7)]
        for i in range(7):
            for j in range(7):
                c00 = pl.multiple_of(ya[i] + xa[j], 4)
                c01 = pl.multiple_of(ya[i] + xb[j], 4)
                c10 = pl.multiple_of(yb[i] + xa[j], 4)
                c11 = pl.multiple_of(yb[i] + xb[j], 4)
                v = jnp.maximum(
                    jnp.maximum(tbl[pl.ds(c00, 4), :], tbl[pl.ds(c01, 4), :]),
                    jnp.maximum(tbl[pl.ds(c10, 4), :], tbl[pl.ds(c11, 4), :]))
                out[i * 7 + j, 0, rr, :, :] = v


def _mm1_kernel(x, w1, b1r, out, acc):
    k = pl.program_id(1)

    @pl.when(k == 0)
    def _():
        acc[...] = jnp.zeros_like(acc)

    acc[...] += jnp.dot(x[...], w1[...], preferred_element_type=jnp.float32)

    @pl.when(k == 13)
    def _():
        out[...] = acc[...] + b1r[...]


def _mm2_kernel(h1b, w2, b2r, wbb, wsb, ocb, ocs, acc):
    k = pl.program_id(1)

    @pl.when(k == 0)
    def _():
        acc[...] = jnp.zeros_like(acc)

    acc[...] += jnp.dot(h1b[...], w2[...],
                        preferred_element_type=jnp.float32)

    @pl.when(k == 3)
    def _():
        x2h = acc[...] + b2r[...]
        ocb[0] = jnp.dot(x2h, wbb[...], preferred_element_type=jnp.float32)
        ocs[0] = jnp.dot(x2h, wsb[...], preferred_element_type=jnp.float32)


def _loss_kernel(pcb, pcs, bbr, bsr, onehotv, selpos, gt84, out):
    cb = pcb[0] + pcb[1] + bbr[...]
    cs = pcs[0] + pcs[1] + bsr[...]
    m = jnp.max(cs, axis=-1, keepdims=True)
    sh = cs - m
    lse = jnp.log(jnp.sum(jnp.exp(sh), axis=-1, keepdims=True))
    logp = sh - lse
    cn = jnp.sum(onehotv[...] * (-logp), axis=(0, 1), keepdims=True)
    cd = jnp.maximum(jnp.sum(onehotv[...], axis=(0, 1), keepdims=True), 1.0)
    cls_loss = cn / cd
    d = jnp.abs(gt84[...] - cb)
    sl1 = jnp.where(d < 1.0, 0.5 * d * d, d - 0.5)
    bbox_loss = jnp.sum(sl1 * selpos[...], axis=(0, 1), keepdims=True)
    out[...] = jnp.concatenate(
        [cls_loss, bbox_loss, cls_loss + 10.0 * bbox_loss], axis=1)


def kernel(base_feature, rois, gt_roi_label, gt_roi_bbox,
           W1, b1, W2, b2, Wb, bb, Ws, bs):
    B, C, H, W = base_feature.shape
    R = rois.shape[1]
    M = B * R
    K1 = Wb.shape[1]
    K = Ws.shape[1]

    feat = pl.pallas_call(
        _tr_kernel,
        grid=(B,),
        in_specs=[pl.BlockSpec((1, C, H, W), lambda b: (b, 0, 0, 0))],
        out_specs=pl.BlockSpec((1, _HW, 4, 128), lambda b: (b, 0, 0, 0)),
        out_shape=jax.ShapeDtypeStruct((B, _HW, 4, 128), jnp.float32),
        compiler_params=pltpu.CompilerParams(
            dimension_semantics=("parallel",),
            vmem_limit_bytes=32 * 1024 * 1024),
    )(base_feature).reshape(B, _TR, 128)

    x1 = rois[..., 0]
    y1 = rois[..., 1]
    x2 = rois[..., 2]
    y2 = rois[..., 3]
    w = x2 - x1 + 1
    h = y2 - y1 + 1
    ii = jnp.arange(7, dtype=jnp.int32)

    rs = y1[..., None] + (ii * h[..., None]) // 7
    re = y1[..., None] + ((ii + 1) * h[..., None] + 6) // 7
    hb = re - rs
    ky = (hb > 2).astype(jnp.int32) + (hb > 4).astype(jnp.int32)
    ya = ky * (3 * _TR) + rs * 200
    yb = ky * (3 * _TR) + (re - (1 << ky)) * 200
    yidx = jnp.stack([ya, yb], axis=-1).astype(jnp.int32).reshape(B, 1, R * 14)

    cs_ = x1[..., None] + (ii * w[..., None]) // 7
    ce_ = x1[..., None] + ((ii + 1) * w[..., None] + 6) // 7
    wb = ce_ - cs_
    kx = (wb > 2).astype(jnp.int32) + (wb > 4).astype(jnp.int32)
    xa = kx * _TR + cs_ * 4
    xb = kx * _TR + (ce_ - (1 << kx)) * 4
    xidx = jnp.stack([xa, xb], axis=-1).astype(jnp.int32).reshape(B, 1, R * 14)

    pooled = pl.pallas_call(
        _pool_kernel,
        grid=(B, _NSTEP),
        in_specs=[
            pl.BlockSpec((1, 1, R * 14), lambda b, s: (b, 0, 0),
                         memory_space=pltpu.SMEM),
            pl.BlockSpec((1, 1, R * 14), lambda b, s: (b, 0, 0),
                         memory_space=pltpu.SMEM),
            pl.BlockSpec((1, _TR, 128), lambda b, s: (b, 0, 0)),
        ],
        out_specs=pl.BlockSpec((49, 1, _NR, 4, 128),
                               lambda b, s: (0, b, s, 0, 0)),
        out_shape=jax.ShapeDtypeStruct((49, B, R, 4, 128), jnp.float32),
        scratch_shapes=[pltpu.VMEM((9 * _TR, 128), jnp.float32)],
        compiler_params=pltpu.CompilerParams(
            dimension_semantics=("parallel", "arbitrary"),
            vmem_limit_bytes=58 * 1024 * 1024),
    )(yidx, xidx, feat)

    xf = pooled.transpose(1, 2, 3, 4, 0).reshape(M, 49 * 512)
    b1r = b1.reshape(1, 4096)

    h1 = pl.pallas_call(
        _mm1_kernel,
        grid=(2, 14),
        in_specs=[
            pl.BlockSpec((M, 1792), lambda nh, k: (0, k)),
            pl.BlockSpec((1792, 2048), lambda nh, k: (k, nh)),
            pl.BlockSpec((1, 2048), lambda nh, k: (0, nh)),
        ],
        out_specs=pl.BlockSpec((M, 2048), lambda nh, k: (0, nh)),
        out_shape=jax.ShapeDtypeStruct((M, 4096), jnp.float32),
        scratch_shapes=[pltpu.VMEM((M, 2048), jnp.float32)],
        compiler_params=pltpu.CompilerParams(
            dimension_semantics=("parallel", "arbitrary"),
            vmem_limit_bytes=50 * 1024 * 1024),
    )(xf, W1, b1r)

    b2r = b2.reshape(1, 4096)
    pcb, pcs = pl.pallas_call(
        _mm2_kernel,
        grid=(2, 4),
        in_specs=[
            pl.BlockSpec((M, 1024), lambda nh, k: (0, k)),
            pl.BlockSpec((1024, 2048), lambda nh, k: (k, nh)),
            pl.BlockSpec((1, 2048), lambda nh, k: (0, nh)),
            pl.BlockSpec((2048, K1), lambda nh, k: (nh, 0)),
            pl.BlockSpec((2048, K), lambda nh, k: (nh, 0)),
        ],
        out_specs=[
            pl.BlockSpec((1, M, K1), lambda nh, k: (nh, 0, 0)),
            pl.BlockSpec((1, M, K), lambda nh, k: (nh, 0, 0)),
        ],
        out_shape=[
            jax.ShapeDtypeStruct((2, M, K1), jnp.float32),
            jax.ShapeDtypeStruct((2, M, K), jnp.float32),
        ],
        scratch_shapes=[pltpu.VMEM((M, 2048), jnp.float32)],
        compiler_params=pltpu.CompilerParams(
            dimension_semantics=("parallel", "arbitrary"),
            vmem_limit_bytes=48 * 1024 * 1024),
    )(h1, W2, b2r, Wb, Ws)

    lblf = gt_roi_label.reshape(M)
    valid = (lblf != -1)
    lbl = jnp.where(valid, lblf, 0)
    onehot = (lbl[:, None] == jnp.arange(K, dtype=jnp.int32)[None, :])
    onehotv = (onehot & valid[:, None]).astype(jnp.float32)
    pos = (lblf > 0).astype(jnp.float32)
    selpos = jnp.repeat(onehot.astype(jnp.float32), 4, axis=1) * pos[:, None]
    gt84 = jnp.tile(gt_roi_bbox.reshape(M, 4), (1, K))

    out3 = pl.pallas_call(
        _loss_kernel,
        grid=(1,),
        in_specs=[
            pl.BlockSpec((2, M, K1), lambda i: (0, 0, 0)),
            pl.BlockSpec((2, M, K), lambda i: (0, 0, 0)),
            pl.BlockSpec((1, K1), lambda i: (0, 0)),
            pl.BlockSpec((1, K), lambda i: (0, 0)),
            pl.BlockSpec((M, K), lambda i: (0, 0)),
            pl.BlockSpec((M, K1), lambda i: (0, 0)),
            pl.BlockSpec((M, K1), lambda i: (0, 0)),
        ],
        out_specs=pl.BlockSpec((1, 3), lambda i: (0, 0)),
        out_shape=jax.ShapeDtypeStruct((1, 3), jnp.float32),
        compiler_params=pltpu.CompilerParams(
            dimension_semantics=("arbitrary",)),
    )(pcb, pcs, bb.reshape(1, K1), bs.reshape(1, K), onehotv, selpos, gt84)

    return out3[0]


# speedup vs baseline: 1.0589x; 1.0486x over previous
import jax
import jax.numpy as jnp
from jax.experimental import pallas as pl
from jax.experimental.pallas import tpu as pltpu

_H, _W, _C = 38, 50, 512
_HW = _H * _W
_TR = _HW * 4
_NR = 8
_NSTEP = 16


def _tr_kernel(src, out):
    for q in range(4):
        out[0, :, q, :] = jnp.transpose(
            src[0, q * 128:(q + 1) * 128, :], (1, 0))


def _pool_kernel(yidx, xidx, feat, out, tbl):
    s = pl.program_id(1)

    @pl.when(s == 0)
    def _build():
        def shmax(dst, src, shift, n):
            tbl[dst:dst + n, :] = jnp.maximum(
                tbl[src:src + n, :], tbl[src + shift:src + shift + n, :])
        tbl[0:_TR, :] = feat[0]
        shmax(_TR, 0, 4, _TR - 4)
        shmax(2 * _TR, _TR, 8, _TR - 8)
        for kx in range(3):
            shmax((3 + kx) * _TR, kx * _TR, 200, _TR - 200)
        for kx in range(3):
            shmax((6 + kx) * _TR, (3 + kx) * _TR, 400, _TR - 400)

    for rr in range(_NR):
        r14 = (s * _NR + rr) * 14
        ya = [yidx[0, 0, r14 + 2 * i] for i in range(7)]
        yb = [yidx[0, 0, r14 + 2 * i + 1] for i in range(7)]
        xa = [xidx[0, 0, r14 + 2 * j] for j in range(7)]
        xb = [xidx[0, 0, r14 + 2 * j + 1] for j in range(7)]
        for i in range(7):
            for j in range(7):
                c00 = pl.multiple_of(ya[i] + xa[j], 4)
                c01 = pl.multiple_of(ya[i] + xb[j], 4)
                c10 = pl.multiple_of(yb[i] + xa[j], 4)
                c11 = pl.multiple_of(yb[i] + xb[j], 4)
                v = jnp.maximum(
                    jnp.maximum(tbl[pl.ds(c00, 4), :], tbl[pl.ds(c01, 4), :]),
                    jnp.maximum(tbl[pl.ds(c10, 4), :], tbl[pl.ds(c11, 4), :]))
                out[i * 7 + j, 0, rr, :, :] = v


def _mm1_kernel(x, w1, b1r, out, acc):
    k = pl.program_id(1)

    @pl.when(k == 0)
    def _():
        acc[...] = jnp.zeros_like(acc)

    acc[...] += jnp.dot(x[...], w1[...], preferred_element_type=jnp.float32)

    @pl.when(k == 13)
    def _():
        out[...] = acc[...] + b1r[...]


def _mm2_kernel(h1b, w2, b2r, wbb, wsb, ocb, ocs, acc):
    k = pl.program_id(1)

    @pl.when(k == 0)
    def _():
        acc[...] = jnp.zeros_like(acc)

    acc[...] += jnp.dot(h1b[...], w2[...],
                        preferred_element_type=jnp.float32)

    @pl.when(k == 3)
    def _():
        x2h = acc[...] + b2r[...]
        ocb[0] = jnp.dot(x2h, wbb[...], preferred_element_type=jnp.float32)
        ocs[0] = jnp.dot(x2h, wsb[...], preferred_element_type=jnp.float32)


def _loss_kernel(pcb, pcs, bbr, bsr, onehotv, selpos, gt84, out):
    cb = pcb[0] + pcb[1] + bbr[...]
    cs = pcs[0] + pcs[1] + bsr[...]
    m = jnp.max(cs, axis=-1, keepdims=True)
    sh = cs - m
    lse = jnp.log(jnp.sum(jnp.exp(sh), axis=-1, keepdims=True))
    logp = sh - lse
    cn = jnp.sum(onehotv[...] * (-logp), axis=(0, 1), keepdims=True)
    cd = jnp.maximum(jnp.sum(onehotv[...], axis=(0, 1), keepdims=True), 1.0)
    cls_loss = cn / cd
    d = jnp.abs(gt84[...] - cb)
    sl1 = jnp.where(d < 1.0, 0.5 * d * d, d - 0.5)
    bbox_loss = jnp.sum(sl1 * selpos[...], axis=(0, 1), keepdims=True)
    out[...] = jnp.concatenate(
        [cls_loss, bbox_loss, cls_loss + 10.0 * bbox_loss], axis=1)


def kernel(base_feature, rois, gt_roi_label, gt_roi_bbox,
           W1, b1, W2, b2, Wb, bb, Ws, bs):
    B, C, H, W = base_feature.shape
    R = rois.shape[1]
    M = B * R
    K1 = Wb.shape[1]
    K = Ws.shape[1]

    feat = pl.pallas_call(
        _tr_kernel,
        grid=(B,),
        in_specs=[pl.BlockSpec((1, C, H * W), lambda b: (b, 0, 0))],
        out_specs=pl.BlockSpec((1, _HW, 4, 128), lambda b: (b, 0, 0, 0)),
        out_shape=jax.ShapeDtypeStruct((B, _HW, 4, 128), jnp.float32),
        compiler_params=pltpu.CompilerParams(
            dimension_semantics=("parallel",),
            vmem_limit_bytes=32 * 1024 * 1024),
    )(base_feature.reshape(B, C, H * W)).reshape(B, _TR, 128)

    x1 = rois[..., 0]
    y1 = rois[..., 1]
    x2 = rois[..., 2]
    y2 = rois[..., 3]
    w = x2 - x1 + 1
    h = y2 - y1 + 1
    ii = jnp.arange(7, dtype=jnp.int32)

    rs = y1[..., None] + (ii * h[..., None]) // 7
    re = y1[..., None] + ((ii + 1) * h[..., None] + 6) // 7
    hb = re - rs
    ky = (hb > 2).astype(jnp.int32) + (hb > 4).astype(jnp.int32)
    ya = ky * (3 * _TR) + rs * 200
    yb = ky * (3 * _TR) + (re - (1 << ky)) * 200
    yidx = jnp.stack([ya, yb], axis=-1).astype(jnp.int32).reshape(B, 1, R * 14)

    cs_ = x1[..., None] + (ii * w[..., None]) // 7
    ce_ = x1[..., None] + ((ii + 1) * w[..., None] + 6) // 7
    wb = ce_ - cs_
    kx = (wb > 2).astype(jnp.int32) + (wb > 4).astype(jnp.int32)
    xa = kx * _TR + cs_ * 4
    xb = kx * _TR + (ce_ - (1 << kx)) * 4
    xidx = jnp.stack([xa, xb], axis=-1).astype(jnp.int32).reshape(B, 1, R * 14)

    pooled = pl.pallas_call(
        _pool_kernel,
        grid=(B, _NSTEP),
        in_specs=[
            pl.BlockSpec((1, 1, R * 14), lambda b, s: (b, 0, 0),
                         memory_space=pltpu.SMEM),
            pl.BlockSpec((1, 1, R * 14), lambda b, s: (b, 0, 0),
                         memory_space=pltpu.SMEM),
            pl.BlockSpec((1, _TR, 128), lambda b, s: (b, 0, 0)),
        ],
        out_specs=pl.BlockSpec((49, 1, _NR, 4, 128),
                               lambda b, s: (0, b, s, 0, 0)),
        out_shape=jax.ShapeDtypeStruct((49, B, R, 4, 128), jnp.float32),
        scratch_shapes=[pltpu.VMEM((9 * _TR, 128), jnp.float32)],
        compiler_params=pltpu.CompilerParams(
            dimension_semantics=("parallel", "arbitrary"),
            vmem_limit_bytes=58 * 1024 * 1024),
    )(yidx, xidx, feat)

    xf = pooled.transpose(1, 2, 3, 4, 0).reshape(M, 49 * 512)
    b1r = b1.reshape(1, 4096)

    h1 = pl.pallas_call(
        _mm1_kernel,
        grid=(2, 14),
        in_specs=[
            pl.BlockSpec((M, 1792), lambda nh, k: (0, k)),
            pl.BlockSpec((1792, 2048), lambda nh, k: (k, nh)),
            pl.BlockSpec((1, 2048), lambda nh, k: (0, nh)),
        ],
        out_specs=pl.BlockSpec((M, 2048), lambda nh, k: (0, nh)),
        out_shape=jax.ShapeDtypeStruct((M, 4096), jnp.float32),
        scratch_shapes=[pltpu.VMEM((M, 2048), jnp.float32)],
        compiler_params=pltpu.CompilerParams(
            dimension_semantics=("parallel", "arbitrary"),
            vmem_limit_bytes=50 * 1024 * 1024),
    )(xf, W1, b1r)

    b2r = b2.reshape(1, 4096)
    pcb, pcs = pl.pallas_call(
        _mm2_kernel,
        grid=(2, 4),
        in_specs=[
            pl.BlockSpec((M, 1024), lambda nh, k: (0, k)),
            pl.BlockSpec((1024, 2048), lambda nh, k: (k, nh)),
            pl.BlockSpec((1, 2048), lambda nh, k: (0, nh)),
            pl.BlockSpec((2048, K1), lambda nh, k: (nh, 0)),
            pl.BlockSpec((2048, K), lambda nh, k: (nh, 0)),
        ],
        out_specs=[
            pl.BlockSpec((1, M, K1), lambda nh, k: (nh, 0, 0)),
            pl.BlockSpec((1, M, K), lambda nh, k: (nh, 0, 0)),
        ],
        out_shape=[
            jax.ShapeDtypeStruct((2, M, K1), jnp.float32),
            jax.ShapeDtypeStruct((2, M, K), jnp.float32),
        ],
        scratch_shapes=[pltpu.VMEM((M, 2048), jnp.float32)],
        compiler_params=pltpu.CompilerParams(
            dimension_semantics=("parallel", "arbitrary"),
            vmem_limit_bytes=48 * 1024 * 1024),
    )(h1, W2, b2r, Wb, Ws)

    lblf = gt_roi_label.reshape(M)
    valid = (lblf != -1)
    lbl = jnp.where(valid, lblf, 0)
    onehot = (lbl[:, None] == jnp.arange(K, dtype=jnp.int32)[None, :])
    onehotv = (onehot & valid[:, None]).astype(jnp.float32)
    pos = (lblf > 0).astype(jnp.float32)
    selpos = jnp.repeat(onehot.astype(jnp.float32), 4, axis=1) * pos[:, None]
    gt84 = jnp.tile(gt_roi_bbox.reshape(M, 4), (1, K))

    out3 = pl.pallas_call(
        _loss_kernel,
        grid=(1,),
        in_specs=[
            pl.BlockSpec((2, M, K1), lambda i: (0, 0, 0)),
            pl.BlockSpec((2, M, K), lambda i: (0, 0, 0)),
            pl.BlockSpec((1, K1), lambda i: (0, 0)),
            pl.BlockSpec((1, K), lambda i: (0, 0)),
            pl.BlockSpec((M, K), lambda i: (0, 0)),
            pl.BlockSpec((M, K1), lambda i: (0, 0)),
            pl.BlockSpec((M, K1), lambda i: (0, 0)),
        ],
        out_specs=pl.BlockSpec((1, 3), lambda i: (0, 0)),
        out_shape=jax.ShapeDtypeStruct((1, 3), jnp.float32),
        compiler_params=pltpu.CompilerParams(
            dimension_semantics=("arbitrary",)),
    )(pcb, pcs, bb.reshape(1, K1), bs.reshape(1, K), onehotv, selpos, gt84)

    return out3[0]
